# baseline (speedup 1.0000x reference)
import jax
import jax.numpy as jnp
from jax.experimental import pallas as pl
from jax.experimental.pallas import tpu as pltpu

_B, _L, _C, _D = 4, 8192, 768, 768
_K = _L // 2
_EPS = 1e-5
_BL = 1024
_NL = _L // _BL
_RING = 4
_INV_SQRT2 = 0.7071067811865476


def _sortable_u32(s):
    u = jax.lax.bitcast_convert_type(s, jnp.uint32)
    neg = (u >> 31) == jnp.uint32(1)
    return jnp.where(neg, ~u, u | jnp.uint32(0x80000000))


def _select_weights(s):
    u = _sortable_u32(s)

    def body(i, acc):
        cand = acc | (jnp.uint32(1) << (31 - i))
        cnt = jnp.sum((u >= cand).astype(jnp.int32))
        return jnp.where(cnt >= _K, cand, acc)

    t = jax.lax.fori_loop(0, 32, body, jnp.uint32(0))
    gt = u > t
    eq = u == t
    need = _K - jnp.sum(gt.astype(jnp.int32))
    idx = (jax.lax.broadcasted_iota(jnp.int32, (_NL, _BL), 0) * _BL
           + jax.lax.broadcasted_iota(jnp.int32, (_NL, _BL), 1))
    eqi = eq.astype(jnp.int32)

    def body2(_, lohi):
        lo, hi = lohi
        mid = (lo + hi) // 2
        cnt = jnp.sum(jnp.where(idx <= mid, eqi, 0))
        ok = cnt >= need
        return jnp.where(ok, lo, mid + 1), jnp.where(ok, mid, hi)

    lo, _ = jax.lax.fori_loop(0, 13, body2, (jnp.int32(0), jnp.int32(_L - 1)))
    sel = gt | (eq & (idx <= lo))
    return sel.astype(jnp.float32) * jnp.float32(1.0 / _K)


def _mega_kernel(x_ref, sw_ref, sW_ref, sb_ref, fW_ref, fb_ref, g_ref, b_ref,
                 o_ref, xstage, x16, sc_scr, w_scr, agg_scr, ostage, sin, sout):
    s = pl.program_id(0)
    p = pl.program_id(1)
    l = pl.program_id(2)

    def xcopy(batch, li):
        return pltpu.make_async_copy(
            x_ref.at[batch, pl.ds(li * _BL, _BL)], xstage.at[li % _RING],
            sin.at[li % _RING])

    def ocopy(batch, li):
        return pltpu.make_async_copy(
            ostage.at[li % 2], o_ref.at[batch, pl.ds(li * _BL, _BL)],
            sout.at[li % 2])

    @pl.when(p == 0)
    def _phase_a():
        @pl.when((s == 0) & (l == 0))
        def _():
            for li in range(_RING):
                xcopy(0, li).start()

        @pl.when((s >= 1) & (l == 0))
        def _():
            w_scr[...] = _select_weights(sc_scr[(s - 1) % 2])
            agg_scr[...] = jnp.zeros_like(agg_scr)

        @pl.when(s < _B)
        def _load():
            xcopy(s, l).wait()
            xb = xstage[l % _RING]
            sc_scr[s % 2, l] = jax.lax.dot_general(
                sw_ref[...], xb, (((1,), (1,)), ((), ())),
                preferred_element_type=jnp.float32)[0]
            x16[s % 2, l] = xb.astype(jnp.bfloat16)

            @pl.when(l + _RING < _NL)
            def _():
                xcopy(s, l + _RING).start()

        @pl.when(s >= 1)
        def _agg():
            feat = jnp.dot(x16[(s - 1) % 2, l], sW_ref[...],
                           preferred_element_type=jnp.float32)
            f16 = (feat + sb_ref[...]).astype(jnp.bfloat16)
            half = jnp.bfloat16(0.5)
            one = jnp.bfloat16(1.0)
            g16 = half * f16 * (one + jax.lax.erf(f16 * jnp.bfloat16(_INV_SQRT2)))
            wblk = w_scr[l][None, :].astype(jnp.bfloat16)
            agg_scr[...] += jnp.dot(wblk, g16,
                                    preferred_element_type=jnp.float32)

    @pl.when(p == 1)
    def _phase_b():
        @pl.when((l < _RING) & (s < _B - 1))
        def _():
            xcopy(s + 1, l).start()

        @pl.when(s >= 1)
        def _store():
            @pl.when(l >= 2)
            def _():
                ocopy(s - 1, l - 2).wait()

            o = jnp.dot(x16[(s - 1) % 2, l], fW_ref[...],
                        preferred_element_type=jnp.float32)
            o = o + fb_ref[...] + agg_scr[...]
            mu = jnp.mean(o, axis=1, keepdims=True)
            d = o - mu
            var = jnp.mean(d * d, axis=1, keepdims=True)
            ostage[l % 2] = (d * jax.lax.rsqrt(var + _EPS) * g_ref[...]
                             + b_ref[...])
            ocopy(s - 1, l).start()

            @pl.when(l == _NL - 1)
            def _():
                ocopy(s - 1, l - 1).wait()
                ocopy(s - 1, l).wait()


@jax.jit
def kernel(x, score_W, score_b, sparse_W, sparse_b, full_W, full_b, gamma, beta):
    del score_b
    sw_row = score_W[:, 0][None, :]
    sW = sparse_W.astype(jnp.bfloat16)
    fW = full_W.astype(jnp.bfloat16)

    return pl.pallas_call(
        _mega_kernel,
        grid=(_B + 1, 2, _NL),
        in_specs=[
            pl.BlockSpec(memory_space=pl.ANY),
            pl.BlockSpec((1, _C), lambda i, p, j: (0, 0)),
            pl.BlockSpec((_C, _D), lambda i, p, j: (0, 0)),
            pl.BlockSpec((1, _D), lambda i, p, j: (0, 0)),
            pl.BlockSpec((_C, _D), lambda i, p, j: (0, 0)),
            pl.BlockSpec((1, _D), lambda i, p, j: (0, 0)),
            pl.BlockSpec((1, _D), lambda i, p, j: (0, 0)),
            pl.BlockSpec((1, _D), lambda i, p, j: (0, 0)),
        ],
        out_specs=pl.BlockSpec(memory_space=pl.ANY),
        out_shape=jax.ShapeDtypeStruct((_B, _L, _D), jnp.float32),
        scratch_shapes=[
            pltpu.VMEM((_RING, _BL, _C), jnp.float32),
            pltpu.VMEM((2, _NL, _BL, _C), jnp.bfloat16),
            pltpu.VMEM((2, _NL, _BL), jnp.float32),
            pltpu.VMEM((_NL, _BL), jnp.float32),
            pltpu.VMEM((1, _D), jnp.float32),
            pltpu.VMEM((2, _BL, _D), jnp.float32),
            pltpu.SemaphoreType.DMA((_RING,)),
            pltpu.SemaphoreType.DMA((2,)),
        ],
        compiler_params=pltpu.CompilerParams(
            dimension_semantics=("arbitrary", "arbitrary", "arbitrary")),
    )(x, sw_row, sW, sparse_b[None, :], fW, full_b[None, :],
      gamma[None, :], beta[None, :])


# speedup vs baseline: 1.0020x; 1.0020x over previous
import jax
import jax.numpy as jnp
from jax.experimental import pallas as pl
from jax.experimental.pallas import tpu as pltpu

_B, _L, _C, _D = 4, 8192, 768, 768
_K = _L // 2
_EPS = 1e-5
_BL = 1024
_NL = _L // _BL
_RING = 3
_INV_SQRT2 = 0.7071067811865476


def _sortable_u32(s):
    u = jax.lax.bitcast_convert_type(s, jnp.uint32)
    neg = (u >> 31) == jnp.uint32(1)
    return jnp.where(neg, ~u, u | jnp.uint32(0x80000000))


def _select_weights(s):
    u = _sortable_u32(s)

    def body(i, acc):
        cand = acc | (jnp.uint32(1) << (31 - i))
        cnt = jnp.sum((u >= cand).astype(jnp.int32))
        return jnp.where(cnt >= _K, cand, acc)

    t = jax.lax.fori_loop(0, 32, body, jnp.uint32(0))
    gt = u > t
    eq = u == t
    need = _K - jnp.sum(gt.astype(jnp.int32))
    idx = (jax.lax.broadcasted_iota(jnp.int32, (_NL, _BL), 0) * _BL
           + jax.lax.broadcasted_iota(jnp.int32, (_NL, _BL), 1))
    eqi = eq.astype(jnp.int32)

    def body2(_, lohi):
        lo, hi = lohi
        mid = (lo + hi) // 2
        cnt = jnp.sum(jnp.where(idx <= mid, eqi, 0))
        ok = cnt >= need
        return jnp.where(ok, lo, mid + 1), jnp.where(ok, mid, hi)

    lo, _ = jax.lax.fori_loop(0, 13, body2, (jnp.int32(0), jnp.int32(_L - 1)))
    sel = gt | (eq & (idx <= lo))
    return sel.astype(jnp.float32) * jnp.float32(1.0 / _K)


def _mega_kernel(x_ref, sw_ref, sW_ref, sb_ref, fW_ref, fb_ref, g_ref, b_ref,
                 o_ref, xstage, x16, sc_scr, w_scr, agg_scr, ostage, sin, sout):
    s = pl.program_id(0)
    l = pl.program_id(1)

    def xcopy(batch, li):
        return pltpu.make_async_copy(
            x_ref.at[batch, pl.ds(li * _BL, _BL)], xstage.at[li % _RING],
            sin.at[li % _RING])

    def ocopy(batch, li):
        return pltpu.make_async_copy(
            ostage.at[li % 2], o_ref.at[batch, pl.ds(li * _BL, _BL)],
            sout.at[li % 2])

    @pl.when((s == 0) & (l == 0))
    def _():
        for li in range(_RING):
            xcopy(0, li).start()

    @pl.when((s >= 1) & (s <= _B) & (l == 0))
    def _():
        w_scr[...] = _select_weights(sc_scr[(s - 1) % 2])
        agg_scr[(s - 1) % 2] = jnp.zeros_like(agg_scr[0])

    @pl.when(s < _B)
    def _load():
        xcopy(s, l).wait()
        xb = xstage[l % _RING]
        sc_scr[s % 2, l] = jax.lax.dot_general(
            sw_ref[...], xb, (((1,), (1,)), ((), ())),
            preferred_element_type=jnp.float32)[0]
        x16[s % 3, l] = xb.astype(jnp.bfloat16)

        @pl.when(l + _RING < _NL)
        def _():
            xcopy(s, l + _RING).start()

        @pl.when((l + _RING >= _NL) & (s + 1 < _B))
        def _():
            xcopy(s + 1, l + _RING - _NL).start()

    @pl.when((s >= 1) & (s <= _B))
    def _agg():
        feat = jnp.dot(x16[(s - 1) % 3, l], sW_ref[...],
                       preferred_element_type=jnp.float32)
        f16 = (feat + sb_ref[...]).astype(jnp.bfloat16)
        half = jnp.bfloat16(0.5)
        one = jnp.bfloat16(1.0)
        g16 = half * f16 * (one + jax.lax.erf(f16 * jnp.bfloat16(_INV_SQRT2)))
        wblk = w_scr[l][None, :].astype(jnp.bfloat16)
        agg_scr[(s - 1) % 2] += jnp.dot(wblk, g16,
                                        preferred_element_type=jnp.float32)

    @pl.when(s >= 2)
    def _store():
        @pl.when(l >= 2)
        def _():
            ocopy(s - 2, l - 2).wait()

        o = jnp.dot(x16[(s - 2) % 3, l], fW_ref[...],
                    preferred_element_type=jnp.float32)
        o = o + fb_ref[...] + agg_scr[(s - 2) % 2]
        mu = jnp.mean(o, axis=1, keepdims=True)
        d = o - mu
        var = jnp.mean(d * d, axis=1, keepdims=True)
        ostage[l % 2] = (d * jax.lax.rsqrt(var + _EPS) * g_ref[...]
                         + b_ref[...])
        ocopy(s - 2, l).start()

        @pl.when(l == _NL - 1)
        def _():
            ocopy(s - 2, l - 1).wait()
            ocopy(s - 2, l).wait()


@jax.jit
def kernel(x, score_W, score_b, sparse_W, sparse_b, full_W, full_b, gamma, beta):
    del score_b
    sw_row = score_W[:, 0][None, :]
    sW = sparse_W.astype(jnp.bfloat16)
    fW = full_W.astype(jnp.bfloat16)

    return pl.pallas_call(
        _mega_kernel,
        grid=(_B + 2, _NL),
        in_specs=[
            pl.BlockSpec(memory_space=pl.ANY),
            pl.BlockSpec((1, _C), lambda i, j: (0, 0)),
            pl.BlockSpec((_C, _D), lambda i, j: (0, 0)),
            pl.BlockSpec((1, _D), lambda i, j: (0, 0)),
            pl.BlockSpec((_C, _D), lambda i, j: (0, 0)),
            pl.BlockSpec((1, _D), lambda i, j: (0, 0)),
            pl.BlockSpec((1, _D), lambda i, j: (0, 0)),
            pl.BlockSpec((1, _D), lambda i, j: (0, 0)),
        ],
        out_specs=pl.BlockSpec(memory_space=pl.ANY),
        out_shape=jax.ShapeDtypeStruct((_B, _L, _D), jnp.float32),
        scratch_shapes=[
            pltpu.VMEM((_RING, _BL, _C), jnp.float32),
            pltpu.VMEM((3, _NL, _BL, _C), jnp.bfloat16),
            pltpu.VMEM((2, _NL, _BL), jnp.float32),
            pltpu.VMEM((_NL, _BL), jnp.float32),
            pltpu.VMEM((2, 1, _D), jnp.float32),
            pltpu.VMEM((2, _BL, _D), jnp.float32),
            pltpu.SemaphoreType.DMA((_RING,)),
            pltpu.SemaphoreType.DMA((2,)),
        ],
        compiler_params=pltpu.CompilerParams(
            dimension_semantics=("arbitrary", "arbitrary")),
    )(x, sw_row, sW, sparse_b[None, :], fW, full_b[None, :],
      gamma[None, :], beta[None, :])


# speedup vs baseline: 1.0045x; 1.0025x over previous
import jax
import jax.numpy as jnp
from jax.experimental import pallas as pl
from jax.experimental.pallas import tpu as pltpu

_B, _L, _C, _D = 4, 8192, 768, 768
_K = _L // 2
_EPS = 1e-5
_BL = 1024
_NL = _L // _BL
_RING = 3
_INV_SQRT2 = 0.7071067811865476


def _sortable_u32(s):
    u = jax.lax.bitcast_convert_type(s, jnp.uint32)
    neg = (u >> 31) == jnp.uint32(1)
    return jnp.where(neg, ~u, u | jnp.uint32(0x80000000))


def _select_weights(s):
    u = _sortable_u32(s)

    def body(i, acc):
        cand = acc | (jnp.uint32(1) << (31 - i))
        cnt = jnp.sum((u >= cand).astype(jnp.int32))
        return jnp.where(cnt >= _K, cand, acc)

    t = jax.lax.fori_loop(0, 32, body, jnp.uint32(0))
    gt = u > t
    eq = u == t
    need = _K - jnp.sum(gt.astype(jnp.int32))
    idx = (jax.lax.broadcasted_iota(jnp.int32, (_NL, _BL), 0) * _BL
           + jax.lax.broadcasted_iota(jnp.int32, (_NL, _BL), 1))
    eqi = eq.astype(jnp.int32)

    def body2(_, lohi):
        lo, hi = lohi
        mid = (lo + hi) // 2
        cnt = jnp.sum(jnp.where(idx <= mid, eqi, 0))
        ok = cnt >= need
        return jnp.where(ok, lo, mid + 1), jnp.where(ok, mid, hi)

    lo, _ = jax.lax.fori_loop(0, 13, body2, (jnp.int32(0), jnp.int32(_L - 1)))
    sel = gt | (eq & (idx <= lo))
    return sel.astype(jnp.float32) * jnp.float32(1.0 / _K)


def _mega_kernel(x_ref, sw_ref, sW_ref, sb_ref, fW_ref, fb_ref, g_ref, b_ref,
                 o_ref, xstage, x16, sc_scr, w_scr, agg_scr, ostage, sin, sout):
    s = pl.program_id(0)
    l = pl.program_id(1)

    def xcopy(batch, li):
        slot = (batch * _NL + li) % _RING
        return pltpu.make_async_copy(
            x_ref.at[batch, pl.ds(li * _BL, _BL)], xstage.at[slot],
            sin.at[slot])

    def ocopy(batch, li):
        return pltpu.make_async_copy(
            ostage.at[li % 2], o_ref.at[batch, pl.ds(li * _BL, _BL)],
            sout.at[li % 2])

    @pl.when((s == 0) & (l == 0))
    def _():
        for li in range(_RING):
            xcopy(0, li).start()

    @pl.when((s >= 1) & (s <= _B) & (l == 0))
    def _():
        w_scr[...] = _select_weights(sc_scr[(s - 1) % 2])
        agg_scr[(s - 1) % 2] = jnp.zeros_like(agg_scr[0])

    @pl.when(s < _B)
    def _load():
        xcopy(s, l).wait()
        xb = xstage[(s * _NL + l) % _RING]
        sc_scr[s % 2, l] = jax.lax.dot_general(
            sw_ref[...], xb, (((1,), (1,)), ((), ())),
            preferred_element_type=jnp.float32)[0]
        x16[s % 3, l] = xb.astype(jnp.bfloat16)

        @pl.when(l + _RING < _NL)
        def _():
            xcopy(s, l + _RING).start()

        @pl.when((l + _RING >= _NL) & (s + 1 < _B))
        def _():
            xcopy(s + 1, l + _RING - _NL).start()

    @pl.when((s >= 1) & (s <= _B))
    def _agg():
        feat = jnp.dot(x16[(s - 1) % 3, l], sW_ref[...],
                       preferred_element_type=jnp.float32)
        f16 = (feat + sb_ref[...]).astype(jnp.bfloat16)
        half = jnp.bfloat16(0.5)
        one = jnp.bfloat16(1.0)
        g16 = half * f16 * (one + jax.lax.erf(f16 * jnp.bfloat16(_INV_SQRT2)))
        wblk = w_scr[l][None, :].astype(jnp.bfloat16)
        agg_scr[(s - 1) % 2] += jnp.dot(wblk, g16,
                                        preferred_element_type=jnp.float32)

    @pl.when(s >= 2)
    def _store():
        @pl.when(l >= 2)
        def _():
            ocopy(s - 2, l - 2).wait()

        o = jnp.dot(x16[(s - 2) % 3, l], fW_ref[...],
                    preferred_element_type=jnp.float32)
        o = o + fb_ref[...] + agg_scr[(s - 2) % 2]
        mu = jnp.mean(o, axis=1, keepdims=True)
        d = o - mu
        var = jnp.mean(d * d, axis=1, keepdims=True)
        ostage[l % 2] = (d * jax.lax.rsqrt(var + _EPS) * g_ref[...]
                         + b_ref[...])
        ocopy(s - 2, l).start()

        @pl.when(l == _NL - 1)
        def _():
            ocopy(s - 2, l - 1).wait()
            ocopy(s - 2, l).wait()


@jax.jit
def kernel(x, score_W, score_b, sparse_W, sparse_b, full_W, full_b, gamma, beta):
    del score_b
    sw_row = score_W[:, 0][None, :]
    sW = sparse_W.astype(jnp.bfloat16)
    fW = full_W.astype(jnp.bfloat16)

    return pl.pallas_call(
        _mega_kernel,
        grid=(_B + 2, _NL),
        in_specs=[
            pl.BlockSpec(memory_space=pl.ANY),
            pl.BlockSpec((1, _C), lambda i, j: (0, 0)),
            pl.BlockSpec((_C, _D), lambda i, j: (0, 0)),
            pl.BlockSpec((1, _D), lambda i, j: (0, 0)),
            pl.BlockSpec((_C, _D), lambda i, j: (0, 0)),
            pl.BlockSpec((1, _D), lambda i, j: (0, 0)),
            pl.BlockSpec((1, _D), lambda i, j: (0, 0)),
            pl.BlockSpec((1, _D), lambda i, j: (0, 0)),
        ],
        out_specs=pl.BlockSpec(memory_space=pl.ANY),
        out_shape=jax.ShapeDtypeStruct((_B, _L, _D), jnp.float32),
        scratch_shapes=[
            pltpu.VMEM((_RING, _BL, _C), jnp.float32),
            pltpu.VMEM((3, _NL, _BL, _C), jnp.bfloat16),
            pltpu.VMEM((2, _NL, _BL), jnp.float32),
            pltpu.VMEM((_NL, _BL), jnp.float32),
            pltpu.VMEM((2, 1, _D), jnp.float32),
            pltpu.VMEM((2, _BL, _D), jnp.float32),
            pltpu.SemaphoreType.DMA((_RING,)),
            pltpu.SemaphoreType.DMA((2,)),
        ],
        compiler_params=pltpu.CompilerParams(
            dimension_semantics=("arbitrary", "arbitrary")),
    )(x, sw_row, sW, sparse_b[None, :], fW, full_b[None, :],
      gamma[None, :], beta[None, :])
